# Initial kernel scaffold; baseline (speedup 1.0000x reference)
import jax
import jax.numpy as jnp
from jax.experimental import pallas as pl


def kernel(uv, feat_2d, feat_3d, w1, b1, w2, b2, w3, b3):
    raise NotImplementedError("write your pallas kernel here")


# speedup vs baseline: 42.9516x; 42.9516x over previous
import functools

import jax
import jax.numpy as jnp
from jax import lax
from jax.experimental import pallas as pl
from jax.experimental.pallas import tpu as pltpu
from jax.experimental.pallas import tpu_sc as plsc

B, H, W = 4, 96, 160
N = H * W
C = 128
KH, KW = 10, 20
P = KH * KW
DIST = 100.0
PH, PW = H + KH - 1, W + KW - 1

NC, NS = 2, 16
NWORK = NC * NS
ROWS_PER_W = (B * N) // NWORK
CHUNK = 128
NCHUNK = ROWS_PER_W // CHUNK

NB = 1024
NBLK = N // NB


def _pcall(*args, **kwargs):
    return pl.pallas_call(*args, **kwargs)


def _leaky(x):
    return jnp.where(x >= 0, x, 0.1 * x)


def _knn_body(uvpad_ref, idx_ref, si_ref):
    b = pl.program_id(0)
    x_f = lax.broadcasted_iota(jnp.int32, (H, W), 1).astype(jnp.float32)
    y_f = lax.broadcasted_iota(jnp.int32, (H, W), 0).astype(jnp.float32)

    best_d2 = None
    best_p = None
    best_dx = None
    best_dy = None
    for p in range(P):
        dh = p // KW
        dw = p % KW
        cx = uvpad_ref[0, 0, dh:dh + H, dw:dw + W]
        cy = uvpad_ref[0, 1, dh:dh + H, dw:dw + W]
        dx = cx - x_f
        dy = cy - y_f
        d2 = dx * dx + dy * dy
        if p == 0:
            best_d2, best_dx, best_dy = d2, dx, dy
            best_p = jnp.zeros((H, W), jnp.int32)
        else:
            m = d2 < best_d2
            best_d2 = jnp.where(m, d2, best_d2)
            best_p = jnp.where(m, p, best_p)
            best_dx = jnp.where(m, dx, best_dx)
            best_dy = jnp.where(m, dy, best_dy)

    r_i = lax.broadcasted_iota(jnp.int32, (H, W), 0)
    c_i = lax.broadcasted_iota(jnp.int32, (H, W), 1)
    dh_i = best_p // KW - (KH // 2)
    dw_i = best_p % KW - (KW // 2)
    cr = jnp.clip(r_i + dh_i, 0, H - 1)
    cc = jnp.clip(c_i + dw_i, 0, W - 1)
    idx = cr * W + cc
    valid = jnp.sqrt(best_d2) <= DIST
    idx = jnp.where(valid, idx, 0)
    idx_ref[0] = idx + b * N

    u0x = uvpad_ref[0, 0, KH // 2, KW // 2]
    u0y = uvpad_ref[0, 1, KH // 2, KW // 2]
    offx = jnp.where(valid, best_dx, u0x - x_f)
    offy = jnp.where(valid, best_dy, u0y - y_f)
    si_ref[0, 0] = offx
    si_ref[0, 1] = offy
    si_ref[0, 2] = jnp.sqrt(offx * offx + offy * offy)


def _knn(uvpad):
    return _pcall(
        _knn_body,
        grid=(B,),
        in_specs=[pl.BlockSpec((1, 2, PH, PW), lambda b: (b, 0, 0, 0))],
        out_specs=[
            pl.BlockSpec((1, H, W), lambda b: (b, 0, 0)),
            pl.BlockSpec((1, 3, H, W), lambda b: (b, 0, 0, 0)),
        ],
        out_shape=[
            jax.ShapeDtypeStruct((B, H, W), jnp.int32),
            jax.ShapeDtypeStruct((B, 3, H, W), jnp.float32),
        ],
    )(uvpad)


def _sc_gather_body(idx_hbm, table_hbm, out_hbm, idx_v, rows_v, sem0, sem1):
    wid = lax.axis_index("s") * NC + lax.axis_index("c")
    pltpu.sync_copy(idx_hbm.at[wid], idx_v)
    sems = [sem0, sem1]
    copies = [None, None]
    copies[0] = pltpu.async_copy(table_hbm.at[idx_v.at[0]], rows_v.at[0], sems[0])
    base = wid * ROWS_PER_W
    for j in range(NCHUNK):
        if j + 1 < NCHUNK:
            nb = (j + 1) % 2
            copies[nb] = pltpu.async_copy(
                table_hbm.at[idx_v.at[j + 1]], rows_v.at[nb], sems[nb])
        cb = j % 2
        copies[cb].wait()
        pltpu.sync_copy(rows_v.at[cb], out_hbm.at[pl.ds(base + j * CHUNK, CHUNK)])


def _sc_gather(idx, table):
    mesh = plsc.VectorSubcoreMesh(core_axis_name="c", subcore_axis_name="s")
    k = pl.kernel(
        _sc_gather_body,
        out_type=jax.ShapeDtypeStruct((B * N, C), jnp.float32),
        mesh=mesh,
        scratch_types=[
            pltpu.VMEM((NCHUNK, CHUNK), jnp.int32),
            pltpu.VMEM((2, CHUNK, C), jnp.float32),
            pltpu.SemaphoreType.DMA,
            pltpu.SemaphoreType.DMA,
        ],
    )
    return k(idx, table)


def _dense_body(sit_ref, g_ref, w1t_ref, b1_ref, w2t_ref, b2_ref, w3_ref,
                b3_ref, out_ref):
    si = sit_ref[0]
    hid = jnp.dot(si, w1t_ref[...], preferred_element_type=jnp.float32)
    hid = _leaky(hid + b1_ref[...])
    sc = jnp.dot(hid, w2t_ref[...], preferred_element_type=jnp.float32)
    sc = jax.nn.sigmoid(sc + b2_ref[...])
    prod = sc * g_ref[...]
    out = lax.dot_general(w3_ref[...], prod, (((1,), (1,)), ((), ())),
                          preferred_element_type=jnp.float32)
    out_ref[0] = _leaky(out + b3_ref[...])


def _dense(sit, g, w1t, b1r, w2t, b2r, w3, b3r):
    return _pcall(
        _dense_body,
        grid=(B, NBLK),
        in_specs=[
            pl.BlockSpec((1, NB, 3), lambda b, i: (b, i, 0)),
            pl.BlockSpec((NB, C), lambda b, i: (b * NBLK + i, 0)),
            pl.BlockSpec((3, 16), lambda b, i: (0, 0)),
            pl.BlockSpec((1, 16), lambda b, i: (0, 0)),
            pl.BlockSpec((16, C), lambda b, i: (0, 0)),
            pl.BlockSpec((1, C), lambda b, i: (0, 0)),
            pl.BlockSpec((C, C), lambda b, i: (0, 0)),
            pl.BlockSpec((C, 1), lambda b, i: (0, 0)),
        ],
        out_specs=pl.BlockSpec((1, C, NB), lambda b, i: (b, 0, i)),
        out_shape=jax.ShapeDtypeStruct((B, C, N), jnp.float32),
    )(sit, g, w1t, b1r, w2t, b2r, w3, b3r)


@jax.jit
def kernel(uv, feat_2d, feat_3d, w1, b1, w2, b2, w3, b3):
    del feat_2d
    uvpad = jnp.pad(uv, ((0, 0), (0, 0), (KH // 2, KH - 1 - KH // 2),
                         (KW // 2, KW - 1 - KW // 2)), mode='edge')
    idx, si = _knn(uvpad)

    table = feat_3d.reshape(B, C, N).transpose(0, 2, 1).reshape(B * N, C)
    g = _sc_gather(idx.reshape(NWORK, NCHUNK, CHUNK), table)

    sit = si.reshape(B, 3, N).transpose(0, 2, 1)
    out = _dense(sit, g, w1.T, b1.reshape(1, 16), w2.T, b2.reshape(1, C),
                 w3, b3.reshape(C, 1))
    return out.reshape(B, C, H, W)
